# baseline (speedup 1.0000x reference)
import functools

import jax
import jax.numpy as jnp
from jax import lax
from jax.experimental import pallas as pl
from jax.experimental.pallas import tpu as pltpu
from jax.experimental.pallas import tpu_sc as plsc

EDIM_ = 32
D_ = 2 * EDIM_
B_ = 16384
NC_, NS_ = 2, 16
NW_ = NC_ * NS_
BPW_ = B_ // NW_


def _sc_gather(user_table, item_table, user_ids, item_ids):
    mesh = plsc.VectorSubcoreMesh(core_axis_name="c", subcore_axis_name="s")

    @functools.partial(
        pl.kernel,
        mesh=mesh,
        compiler_params=pltpu.CompilerParams(use_tc_tiling_on_sc=False),
        out_type=[
            jax.ShapeDtypeStruct((B_, D_), jnp.float32),
            jax.ShapeDtypeStruct((B_, D_), jnp.float32),
        ],
        scratch_types=[
            pltpu.VMEM((BPW_,), jnp.int32),
            pltpu.VMEM((BPW_,), jnp.int32),
            pltpu.VMEM((BPW_, D_), jnp.float32),
            pltpu.VMEM((BPW_, D_), jnp.float32),
            pltpu.SemaphoreType.DMA,
            pltpu.SemaphoreType.DMA,
        ],
    )
    def gather_kernel(ut_hbm, it_hbm, uid_hbm, iid_hbm, ue_hbm, ie_hbm,
                      uidx_v, iidx_v, ur_v, ir_v, sem_u, sem_i):
        wid = lax.axis_index("s") * NC_ + lax.axis_index("c")
        base = wid * BPW_
        pltpu.sync_copy(uid_hbm.at[pl.ds(base, BPW_)], uidx_v)
        pltpu.sync_copy(iid_hbm.at[pl.ds(base, BPW_)], iidx_v)
        cu = pltpu.async_copy(ut_hbm.at[uidx_v], ur_v, sem_u)
        ci = pltpu.async_copy(it_hbm.at[iidx_v], ir_v, sem_i)
        cu.wait()
        pltpu.sync_copy(ur_v, ue_hbm.at[pl.ds(base, BPW_)])
        ci.wait()
        pltpu.sync_copy(ir_v, ie_hbm.at[pl.ds(base, BPW_)])

    return gather_kernel(user_table, item_table, user_ids, item_ids)


def _tc_head_body(ue_ref, ie_ref, w1_ref, b1_ref, w2_ref, b2_ref,
                  w3_ref, b3_ref, o_ref):
    ue = ue_ref[...]
    ie = ie_ref[...]
    gmf = ue[:, :EDIM_] * ie[:, :EDIM_]
    x = jnp.concatenate([ue[:, EDIM_:], ie[:, EDIM_:]], axis=1)
    h1 = lax.dot_general(x, w1_ref[...], (((1,), (1,)), ((), ())),
                         preferred_element_type=jnp.float32)
    h1 = jnp.maximum(h1 + b1_ref[...], 0.0)
    h2 = lax.dot_general(h1, w2_ref[...], (((1,), (1,)), ((), ())),
                         preferred_element_type=jnp.float32)
    h2 = jnp.maximum(h2 + b2_ref[...], 0.0)
    z = jnp.concatenate([gmf, h2], axis=1)
    o = lax.dot_general(z, w3_ref[...], (((1,), (1,)), ((), ())),
                        preferred_element_type=jnp.float32)
    o_ref[...] = o + b3_ref[0]


def _tc_head(ue, ie, W1, b1, W2, b2, W3, b3):
    out = pl.pallas_call(
        _tc_head_body,
        in_specs=[pl.BlockSpec() for _ in range(7)]
        + [pl.BlockSpec(memory_space=pltpu.MemorySpace.SMEM)],
        out_shape=jax.ShapeDtypeStruct((B_, 8), jnp.float32),
    )(ue, ie, W1, b1.reshape(1, EDIM_), W2, b2.reshape(1, EDIM_ // 2),
      jnp.broadcast_to(W3, (8, EDIM_ + EDIM_ // 2)), b3)
    return out[:, 0]


def kernel(user_ids, item_ids, user_table, item_table, W1, b1, W2, b2, W3, b3):
    uid = user_ids.astype(jnp.int32)
    iid = item_ids.astype(jnp.int32)
    ue, ie = _sc_gather(user_table, item_table, uid, iid)
    return _tc_head(ue, ie, W1, b1, W2, b2, W3, b3)
